# baseline (speedup 1.0000x reference)
import functools

import jax
import jax.numpy as jnp
from jax import lax
from jax.experimental import pallas as pl
from jax.experimental.pallas import tpu as pltpu
from jax.experimental.pallas import tpu_sc as plsc

VOCAB = 1000000
N_NEW = 2000
BATCH = 4096
HIST = 200
D = 32
PADW = 128
SUB = PADW // D

NC, NS, L = 2, 16, 16
NW = NC * NS
BW_ = BATCH // NW
BG = BW_ // L
CH = 8
N_CHUNKS = HIST // CH
ETILES = D // 8


def _body(idx_hbm, tb_hbm, out_hbm,
          idx_v, idxg_v, raw_v, t_v, sem_g0, sem_g1, sem_o):
    cid = lax.axis_index("c")
    sid = lax.axis_index("s")
    wid = sid * NC + cid
    b0 = wid * BW_
    iota = lax.iota(jnp.int32, L)
    sem_g = [sem_g0, sem_g1]

    pltpu.sync_copy(idx_hbm.at[pl.ds(b0, BW_)], idx_v)

    def build_and_fire(c, par):
        def build(i, _):
            hh = i // BG
            bg = i % BG
            b16 = bg * L + iota
            h = c * CH + hh
            vec = plsc.load_gather(idx_v, [b16, jnp.full((L,), 0, jnp.int32) + h])
            idxg_v[par, hh, pl.ds(bg * L, L)] = vec * SUB
            return 0

        lax.fori_loop(0, CH * BG, build, 0)
        return [pltpu.async_copy(tb_hbm.at[idxg_v.at[par, hh]],
                                 raw_v.at[par, hh], sem_g[par])
                for hh in range(CH)]

    def drain_gathers(par):
        for hh in range(CH):
            pltpu.make_async_copy(tb_hbm.at[idxg_v.at[par, hh]],
                                  raw_v.at[par, hh], sem_g[par]).wait()

    build_and_fire(0, 0)

    def process_chunk(c, par):
        drain_gathers(par)

        for hh in range(CH):
            tp = hh % 2
            h = c * CH + hh

            @pl.when(c * CH + hh >= 2)
            def _():
                for te in range(ETILES):
                    pltpu.make_async_copy(
                        t_v.at[tp, pl.ds(te * 8, 8)],
                        out_hbm.at[h, te, wid], sem_o).wait()

            def transpose(cc, _):
                ccv = jnp.full((L,), 0, jnp.int32) + cc
                for bg in range(BG):
                    b16 = bg * L + iota
                    vals = plsc.load_gather(raw_v.at[par, hh], [b16, ccv])
                    t_v[tp, cc, pl.ds(bg * L, L)] = vals
                return 0

            lax.fori_loop(0, D, transpose, 0)

            for te in range(ETILES):
                pltpu.async_copy(t_v.at[tp, pl.ds(te * 8, 8)],
                                 out_hbm.at[h, te, wid], sem_o)

    def chunk_pair(c2, _):
        for k in (0, 1):
            c = c2 * 2 + k

            @pl.when(c < N_CHUNKS)
            def _():
                @pl.when(c + 1 < N_CHUNKS)
                def _():
                    build_and_fire(c + 1, 1 - k)

                process_chunk(c, k)
        return 0

    lax.fori_loop(0, (N_CHUNKS + 1) // 2, chunk_pair, 0)

    for _ in range(2):
        for te in range(ETILES):
            pltpu.make_async_copy(t_v.at[0, pl.ds(te * 8, 8)],
                                  out_hbm.at[0, te, wid], sem_o).wait()


_mesh = plsc.VectorSubcoreMesh(
    core_axis_name="c", subcore_axis_name="s", num_cores=NC, num_subcores=NS)

_emb = functools.partial(
    pl.kernel,
    out_type=jax.ShapeDtypeStruct((HIST, ETILES, NW, 8, PADW), jnp.float32),
    mesh=_mesh,
    scratch_types=[
        pltpu.VMEM((BW_, HIST), jnp.int32),
        pltpu.VMEM((2, CH, BW_), jnp.int32),
        pltpu.VMEM((2, CH, BW_, D), jnp.float32),
        pltpu.VMEM((2, D, PADW), jnp.float32),
        pltpu.SemaphoreType.DMA,
        pltpu.SemaphoreType.DMA,
        pltpu.SemaphoreType.DMA,
    ],
    compiler_params=pltpu.CompilerParams(
        use_tc_tiling_on_sc=False, needs_layout_passes=False),
)(_body)


def kernel(input, table_base, table_new):
    full = jnp.concatenate([table_base, table_new], axis=0)
    fullp = jnp.pad(full, ((0, 0), (0, PADW - D)))
    tb32 = fullp.reshape((VOCAB + N_NEW) * SUB, D)
    out5 = _emb(input.astype(jnp.int32), tb32)
    return out5.transpose(2, 4, 0, 1, 3).reshape(BATCH, HIST, D)
